# Initial kernel scaffold; baseline (speedup 1.0000x reference)
import jax
import jax.numpy as jnp
from jax.experimental import pallas as pl


def kernel(p, x, o, W1, b1, gamma, beta, W2, b2):
    raise NotImplementedError("write your pallas kernel here")


# speedup vs baseline: 4.9528x; 4.9528x over previous
import functools

import jax
import jax.numpy as jnp
from jax.experimental import pallas as pl
from jax.experimental.pallas import tpu as pltpu

N = 32768
B = 16
D = 128
R = 4096
T = N // R

_HI = jax.lax.Precision.HIGHEST


def _onehot(t, o, o_prev, rows):
    r = jax.lax.broadcasted_iota(jnp.int32, (rows, B), 0) + t * rows
    return ((r >= o_prev) & (r < o)).astype(jnp.float32)


def _pass_a(x_ref, o_ref, op_ref, w_ref, segsum_ref, sa2_ref):
    t = pl.program_id(0)

    @pl.when(t == 0)
    def _init():
        segsum_ref[...] = jnp.zeros_like(segsum_ref)
        sa2_ref[...] = jnp.zeros_like(sa2_ref)

    x = x_ref[...]
    a = jnp.dot(x, w_ref[...], preferred_element_type=jnp.float32, precision=_HI)
    oh = _onehot(t, o_ref[...], op_ref[...], R)
    seg = jax.lax.dot_general(oh, x, (((0,), (0,)), ((), ())),
                              preferred_element_type=jnp.float32, precision=_HI)
    segsum_ref[...] += seg
    sa2_ref[...] += jnp.sum(a * a, axis=0, keepdims=True)


def _mid(segsum_ref, sa2_ref, cnt_ref, w1a_ref, w1b_ref, b1_ref, gamma_ref,
         beta_ref, w2_ref, b2_ref, scale_ref, f_ref):
    cnt = cnt_ref[...]
    segsum = segsum_ref[...]
    seg_mean = segsum / jnp.maximum(cnt, 1.0)
    g = jax.nn.relu(jnp.dot(seg_mean, w2_ref[...],
                            preferred_element_type=jnp.float32, precision=_HI)
                    + b2_ref[...])
    e = jnp.dot(g, w1b_ref[...], preferred_element_type=jnp.float32,
                precision=_HI) + b1_ref[...]
    segsum_a = jnp.dot(segsum, w1a_ref[...], preferred_element_type=jnp.float32,
                       precision=_HI)
    sum_h = jnp.sum(segsum_a + cnt * e, axis=0, keepdims=True)
    sum_h2 = sa2_ref[...] + jnp.sum(2.0 * e * segsum_a + cnt * e * e,
                                    axis=0, keepdims=True)
    mu = sum_h / N
    var = sum_h2 / N - mu * mu
    scale = gamma_ref[...] * jax.lax.rsqrt(var + 1e-5)
    shift = beta_ref[...] - mu * scale
    scale_ref[...] = scale
    f_ref[...] = e * scale + shift


def _pass_b(x_ref, o_ref, op_ref, w_ref, scale_ref, f_ref, out_ref):
    t = pl.program_id(0)
    x = x_ref[...]
    a = jnp.dot(x, w_ref[...], preferred_element_type=jnp.float32, precision=_HI)
    oh = _onehot(t, o_ref[...], op_ref[...], R)
    seg_f = jnp.dot(oh, f_ref[...], preferred_element_type=jnp.float32,
                    precision=_HI)
    out_ref[...] = jax.nn.relu(a * scale_ref[...] + seg_f)


def kernel(p, x, o, W1, b1, gamma, beta, W2, b2):
    del p
    o2 = o.reshape(1, B)
    o_prev = jnp.concatenate([jnp.zeros((1, 1), jnp.int32), o2[:, :-1]], axis=1)
    cnt = (o2 - o_prev).reshape(B, 1).astype(jnp.float32)
    W1a = W1[:D]
    W1b = W1[D:]

    full = lambda shape: pl.BlockSpec(shape, lambda *_: (0,) * len(shape))

    segsum, sa2 = pl.pallas_call(
        _pass_a,
        grid=(T,),
        in_specs=[
            pl.BlockSpec((R, D), lambda t: (t, 0)),
            full((1, B)), full((1, B)), full((D, D)),
        ],
        out_specs=[full((B, D)), full((1, D))],
        out_shape=[
            jax.ShapeDtypeStruct((B, D), jnp.float32),
            jax.ShapeDtypeStruct((1, D), jnp.float32),
        ],
    )(x, o2, o_prev, W1a)

    scale, f = pl.pallas_call(
        _mid,
        in_specs=[full((B, D)), full((1, D)), full((B, 1)), full((D, D)),
                  full((D, D)), full((1, D)), full((1, D)), full((1, D)),
                  full((D, D)), full((1, D))],
        out_specs=[full((1, D)), full((B, D))],
        out_shape=[
            jax.ShapeDtypeStruct((1, D), jnp.float32),
            jax.ShapeDtypeStruct((B, D), jnp.float32),
        ],
    )(segsum, sa2, cnt, W1a, W1b, b1.reshape(1, D), gamma.reshape(1, D),
      beta.reshape(1, D), W2, b2.reshape(1, D))

    out = pl.pallas_call(
        _pass_b,
        grid=(T,),
        in_specs=[
            pl.BlockSpec((R, D), lambda t: (t, 0)),
            full((1, B)), full((1, B)), full((D, D)), full((1, D)),
            full((B, D)),
        ],
        out_specs=pl.BlockSpec((R, D), lambda t: (t, 0)),
        out_shape=jax.ShapeDtypeStruct((N, D), jnp.float32),
    )(x, o2, o_prev, W1a, scale, f)
    return out


# speedup vs baseline: 8.0122x; 1.6177x over previous
import functools

import jax
import jax.numpy as jnp
from jax.experimental import pallas as pl
from jax.experimental.pallas import tpu as pltpu

N = 32768
B = 16
D = 128
R = 4096
T = N // R

_HI = jax.lax.Precision.HIGHEST
_BIG = jax.lax.Precision.DEFAULT


def _onehot(t, o, o_prev, rows):
    r = jax.lax.broadcasted_iota(jnp.int32, (rows, B), 0) + t * rows
    return ((r >= o_prev) & (r < o)).astype(jnp.float32)


def _pass_a(x_ref, o_ref, op_ref, w_ref, segsum_ref, sa2_ref):
    t = pl.program_id(0)

    @pl.when(t == 0)
    def _init():
        segsum_ref[...] = jnp.zeros_like(segsum_ref)
        sa2_ref[...] = jnp.zeros_like(sa2_ref)

    x = x_ref[...]
    a = jnp.dot(x, w_ref[...], preferred_element_type=jnp.float32, precision=_BIG)
    oh = _onehot(t, o_ref[...], op_ref[...], R)
    seg = jax.lax.dot_general(oh, x, (((0,), (0,)), ((), ())),
                              preferred_element_type=jnp.float32, precision=_BIG)
    segsum_ref[...] += seg
    sa2_ref[...] += jnp.sum(a * a, axis=0, keepdims=True)


def _mid(segsum_ref, sa2_ref, cnt_ref, w1a_ref, w1b_ref, b1_ref, gamma_ref,
         beta_ref, w2_ref, b2_ref, scale_ref, f_ref):
    cnt = cnt_ref[...]
    segsum = segsum_ref[...]
    seg_mean = segsum / jnp.maximum(cnt, 1.0)
    g = jax.nn.relu(jnp.dot(seg_mean, w2_ref[...],
                            preferred_element_type=jnp.float32, precision=_HI)
                    + b2_ref[...])
    e = jnp.dot(g, w1b_ref[...], preferred_element_type=jnp.float32,
                precision=_HI) + b1_ref[...]
    segsum_a = jnp.dot(segsum, w1a_ref[...], preferred_element_type=jnp.float32,
                       precision=_HI)
    sum_h = jnp.sum(segsum_a + cnt * e, axis=0, keepdims=True)
    sum_h2 = sa2_ref[...] + jnp.sum(2.0 * e * segsum_a + cnt * e * e,
                                    axis=0, keepdims=True)
    mu = sum_h / N
    var = sum_h2 / N - mu * mu
    scale = gamma_ref[...] * jax.lax.rsqrt(var + 1e-5)
    shift = beta_ref[...] - mu * scale
    scale_ref[...] = scale
    f_ref[...] = e * scale + shift


def _pass_b(x_ref, o_ref, op_ref, w_ref, scale_ref, f_ref, out_ref):
    t = pl.program_id(0)
    x = x_ref[...]
    a = jnp.dot(x, w_ref[...], preferred_element_type=jnp.float32, precision=_BIG)
    oh = _onehot(t, o_ref[...], op_ref[...], R)
    seg_f = jnp.dot(oh, f_ref[...], preferred_element_type=jnp.float32,
                    precision=_HI)
    out_ref[...] = jax.nn.relu(a * scale_ref[...] + seg_f)


def kernel(p, x, o, W1, b1, gamma, beta, W2, b2):
    del p
    o2 = o.reshape(1, B)
    o_prev = jnp.concatenate([jnp.zeros((1, 1), jnp.int32), o2[:, :-1]], axis=1)
    cnt = (o2 - o_prev).reshape(B, 1).astype(jnp.float32)
    W1a = W1[:D]
    W1b = W1[D:]

    full = lambda shape: pl.BlockSpec(shape, lambda *_: (0,) * len(shape))

    segsum, sa2 = pl.pallas_call(
        _pass_a,
        grid=(T,),
        in_specs=[
            pl.BlockSpec((R, D), lambda t: (t, 0)),
            full((1, B)), full((1, B)), full((D, D)),
        ],
        out_specs=[full((B, D)), full((1, D))],
        out_shape=[
            jax.ShapeDtypeStruct((B, D), jnp.float32),
            jax.ShapeDtypeStruct((1, D), jnp.float32),
        ],
    )(x, o2, o_prev, W1a)

    scale, f = pl.pallas_call(
        _mid,
        in_specs=[full((B, D)), full((1, D)), full((B, 1)), full((D, D)),
                  full((D, D)), full((1, D)), full((1, D)), full((1, D)),
                  full((D, D)), full((1, D))],
        out_specs=[full((1, D)), full((B, D))],
        out_shape=[
            jax.ShapeDtypeStruct((1, D), jnp.float32),
            jax.ShapeDtypeStruct((B, D), jnp.float32),
        ],
    )(segsum, sa2, cnt, W1a, W1b, b1.reshape(1, D), gamma.reshape(1, D),
      beta.reshape(1, D), W2, b2.reshape(1, D))

    out = pl.pallas_call(
        _pass_b,
        grid=(T,),
        in_specs=[
            pl.BlockSpec((R, D), lambda t: (t, 0)),
            full((1, B)), full((1, B)), full((D, D)), full((1, D)),
            full((B, D)),
        ],
        out_specs=pl.BlockSpec((R, D), lambda t: (t, 0)),
        out_shape=jax.ShapeDtypeStruct((N, D), jnp.float32),
    )(x, o2, o_prev, W1a, scale, f)
    return out


# speedup vs baseline: 10.4443x; 1.3036x over previous
import jax
import jax.numpy as jnp
from jax.experimental import pallas as pl
from jax.experimental.pallas import tpu as pltpu

N = 32768
B = 16
D = 128
R = 4096
T = N // R


def _onehot(t, o, o_prev, rows):
    r = jax.lax.broadcasted_iota(jnp.int32, (rows, B), 0) + t * rows
    return ((r >= o_prev) & (r < o)).astype(jnp.float32)


def _rowdot(lhs, rhs):
    return jax.lax.dot_general(lhs, rhs, (((0,), (0,)), ((), ())),
                               preferred_element_type=jnp.float32)


def _pass_a(x_ref, o_ref, op_ref, g_ref, segsum_ref):
    t = pl.program_id(0)

    @pl.when(t == 0)
    def _init():
        g_ref[...] = jnp.zeros_like(g_ref)
        segsum_ref[...] = jnp.zeros_like(segsum_ref)

    x = x_ref[...]
    oh = _onehot(t, o_ref[...], op_ref[...], R)
    g_ref[...] += _rowdot(x, x)
    segsum_ref[...] += _rowdot(oh, x)


def _pass_b(x_ref, o_ref, op_ref, cnt_ref, gram_ref, segsum_ref, w1a_ref,
            w1b_ref, b1_ref, gamma_ref, beta_ref, w2_ref, b2_ref, out_ref,
            scale_ref, f_ref):
    t = pl.program_id(0)

    @pl.when(t == 0)
    def _mid():
        cnt = cnt_ref[...]
        segsum = segsum_ref[...]
        w1a = w1a_ref[...]
        seg_mean = segsum / jnp.maximum(cnt, 1.0)
        g = jax.nn.relu(jnp.dot(seg_mean, w2_ref[...],
                                preferred_element_type=jnp.float32)
                        + b2_ref[...])
        e = jnp.dot(g, w1b_ref[...],
                    preferred_element_type=jnp.float32) + b1_ref[...]
        segsum_a = jnp.dot(segsum, w1a, preferred_element_type=jnp.float32)
        sum_a2 = jnp.sum(w1a * jnp.dot(gram_ref[...], w1a,
                                       preferred_element_type=jnp.float32),
                         axis=0, keepdims=True)
        sum_h = jnp.sum(segsum_a + cnt * e, axis=0, keepdims=True)
        sum_h2 = sum_a2 + jnp.sum(2.0 * e * segsum_a + cnt * e * e,
                                  axis=0, keepdims=True)
        mu = sum_h / N
        var = sum_h2 / N - mu * mu
        scale = gamma_ref[...] * jax.lax.rsqrt(var + 1e-5)
        shift = beta_ref[...] - mu * scale
        scale_ref[...] = scale
        f_ref[...] = e * scale + shift

    x = x_ref[...]
    a = jnp.dot(x, w1a_ref[...], preferred_element_type=jnp.float32)
    oh = _onehot(t, o_ref[...], op_ref[...], R)
    seg_f = jnp.dot(oh, f_ref[...], preferred_element_type=jnp.float32)
    out_ref[...] = jax.nn.relu(a * scale_ref[...] + seg_f)


def kernel(p, x, o, W1, b1, gamma, beta, W2, b2):
    del p
    o2 = o.reshape(1, B)
    o_prev = jnp.concatenate([jnp.zeros((1, 1), jnp.int32), o2[:, :-1]], axis=1)
    cnt = (o2 - o_prev).reshape(B, 1).astype(jnp.float32)
    W1a = W1[:D]
    W1b = W1[D:]

    full = lambda shape: pl.BlockSpec(shape, lambda *_: (0,) * len(shape))

    gram, segsum = pl.pallas_call(
        _pass_a,
        grid=(T,),
        in_specs=[
            pl.BlockSpec((R, D), lambda t: (t, 0)),
            full((1, B)), full((1, B)),
        ],
        out_specs=[full((D, D)), full((B, D))],
        out_shape=[
            jax.ShapeDtypeStruct((D, D), jnp.float32),
            jax.ShapeDtypeStruct((B, D), jnp.float32),
        ],
    )(x, o2, o_prev)

    out = pl.pallas_call(
        _pass_b,
        grid=(T,),
        in_specs=[
            pl.BlockSpec((R, D), lambda t: (t, 0)),
            full((1, B)), full((1, B)), full((B, 1)), full((D, D)),
            full((B, D)), full((D, D)), full((D, D)), full((1, D)),
            full((1, D)), full((1, D)), full((D, D)), full((1, D)),
        ],
        out_specs=pl.BlockSpec((R, D), lambda t: (t, 0)),
        out_shape=jax.ShapeDtypeStruct((N, D), jnp.float32),
        scratch_shapes=[
            pltpu.VMEM((1, D), jnp.float32),
            pltpu.VMEM((B, D), jnp.float32),
        ],
    )(x, o2, o_prev, cnt, gram, segsum, W1a, W1b, b1.reshape(1, D),
      gamma.reshape(1, D), beta.reshape(1, D), W2, b2.reshape(1, D))
    return out


# speedup vs baseline: 11.8441x; 1.1340x over previous
import jax
import jax.numpy as jnp
from jax.experimental import pallas as pl
from jax.experimental.pallas import tpu as pltpu

N = 32768
B = 16
D = 128
R = 4096
T = N // R


def _body(x_ref, o_ref, w1_ref, b1_ref, gamma_ref, beta_ref, w2_ref, b2_ref,
          out_ref, gram_ref, segsum_ref, scale_ref, f_ref):
    i = pl.program_id(0)
    phase_a = i < T
    t = jnp.where(phase_a, i, i - T)

    o_col = o_ref[...]
    op_col = jnp.concatenate(
        [jnp.zeros((1, 1), jnp.int32), o_col[:-1, :]], axis=0)
    r = jax.lax.broadcasted_iota(jnp.int32, (B, R), 1) + t * R
    oh_t = ((r >= op_col) & (r < o_col)).astype(jnp.float32)

    x = x_ref[...]

    @pl.when(phase_a)
    def _accum():
        @pl.when(i == 0)
        def _init():
            gram_ref[...] = jnp.zeros_like(gram_ref)
            segsum_ref[...] = jnp.zeros_like(segsum_ref)

        gram_ref[...] += jax.lax.dot_general(
            x, x, (((0,), (0,)), ((), ())), preferred_element_type=jnp.float32)
        segsum_ref[...] += jnp.dot(oh_t, x, preferred_element_type=jnp.float32)

    @pl.when(i == T)
    def _mid():
        cnt = (o_col - op_col).astype(jnp.float32)
        segsum = segsum_ref[...]
        w1a = w1_ref[:D, :]
        seg_mean = segsum / jnp.maximum(cnt, 1.0)
        g = jax.nn.relu(jnp.dot(seg_mean, w2_ref[...],
                                preferred_element_type=jnp.float32)
                        + b2_ref[...])
        e = jnp.dot(g, w1_ref[D:, :],
                    preferred_element_type=jnp.float32) + b1_ref[...]
        segsum_a = jnp.dot(segsum, w1a, preferred_element_type=jnp.float32)
        sum_a2 = jnp.sum(w1a * jnp.dot(gram_ref[...], w1a,
                                       preferred_element_type=jnp.float32),
                         axis=0, keepdims=True)
        sum_h = jnp.sum(segsum_a + cnt * e, axis=0, keepdims=True)
        sum_h2 = sum_a2 + jnp.sum(2.0 * e * segsum_a + cnt * e * e,
                                  axis=0, keepdims=True)
        mu = sum_h / N
        var = sum_h2 / N - mu * mu
        scale = gamma_ref[...] * jax.lax.rsqrt(var + 1e-5)
        shift = beta_ref[...] - mu * scale
        scale_ref[...] = scale
        f_ref[...] = e * scale + shift

    @pl.when(jnp.logical_not(phase_a))
    def _apply():
        a = jnp.dot(x, w1_ref[:D, :], preferred_element_type=jnp.float32)
        seg_f = jax.lax.dot_general(
            oh_t, f_ref[...], (((0,), (0,)), ((), ())),
            preferred_element_type=jnp.float32)
        out_ref[...] = jax.nn.relu(a * scale_ref[...] + seg_f)


def kernel(p, x, o, W1, b1, gamma, beta, W2, b2):
    del p
    full = lambda shape: pl.BlockSpec(shape, lambda *_: (0,) * len(shape))
    x_spec = pl.BlockSpec((R, D), lambda i: (jnp.where(i < T, i, i - T), 0))
    out_spec = pl.BlockSpec((R, D), lambda i: (jnp.where(i < T, 0, i - T), 0))

    return pl.pallas_call(
        _body,
        grid=(2 * T,),
        in_specs=[
            x_spec,
            full((B, 1)), full((2 * D, D)), full((1, D)), full((1, D)),
            full((1, D)), full((D, D)), full((1, D)),
        ],
        out_specs=out_spec,
        out_shape=jax.ShapeDtypeStruct((N, D), jnp.float32),
        scratch_shapes=[
            pltpu.VMEM((D, D), jnp.float32),
            pltpu.VMEM((B, D), jnp.float32),
            pltpu.VMEM((1, D), jnp.float32),
            pltpu.VMEM((B, D), jnp.float32),
        ],
    )(x, o.reshape(B, 1), W1, b1.reshape(1, D), gamma.reshape(1, D),
      beta.reshape(1, D), W2, b2.reshape(1, D))


# speedup vs baseline: 14.3965x; 1.2155x over previous
import jax
import jax.numpy as jnp
from jax.experimental import pallas as pl
from jax.experimental.pallas import tpu as pltpu

N = 32768
B = 16
D = 128
R = 4096
T = N // R


def _body(x_ref, o_ref, w1_ref, b1_ref, gamma_ref, beta_ref, w2_ref, b2_ref,
          out_ref, gram_ref, segsum_ref, scale_ref, f_ref, xbuf_ref):
    i = pl.program_id(0)
    phase_a = i < T
    t = jnp.where(phase_a, i, i - T)

    o_col = o_ref[...]
    op_col = jnp.concatenate(
        [jnp.zeros((1, 1), jnp.int32), o_col[:-1, :]], axis=0)
    base = t * R
    r = jax.lax.broadcasted_iota(jnp.int32, (B, R), 1)
    oh_t = ((r >= op_col - base) & (r < o_col - base)).astype(jnp.float32)

    @pl.when(phase_a)
    def _accum():
        @pl.when(i == 0)
        def _init():
            gram_ref[...] = jnp.zeros_like(gram_ref)
            segsum_ref[...] = jnp.zeros_like(segsum_ref)

        x = x_ref[...]
        xbuf_ref[pl.ds(i * R, R), :] = x
        gram_ref[...] += jax.lax.dot_general(
            x, x, (((0,), (0,)), ((), ())), preferred_element_type=jnp.float32)
        segsum_ref[...] += jnp.dot(oh_t, x, preferred_element_type=jnp.float32)

    @pl.when(i == T)
    def _mid():
        cnt = (o_col - op_col).astype(jnp.float32)
        segsum = segsum_ref[...]
        w1a = w1_ref[:D, :]
        seg_mean = segsum / jnp.maximum(cnt, 1.0)
        g = jax.nn.relu(jnp.dot(seg_mean, w2_ref[...],
                                preferred_element_type=jnp.float32)
                        + b2_ref[...])
        e = jnp.dot(g, w1_ref[D:, :],
                    preferred_element_type=jnp.float32) + b1_ref[...]
        segsum_a = jnp.dot(segsum, w1a, preferred_element_type=jnp.float32)
        sum_a2 = jnp.sum(w1a * jnp.dot(gram_ref[...], w1a,
                                       preferred_element_type=jnp.float32),
                         axis=0, keepdims=True)
        sum_h = jnp.sum(segsum_a + cnt * e, axis=0, keepdims=True)
        sum_h2 = sum_a2 + jnp.sum(2.0 * e * segsum_a + cnt * e * e,
                                  axis=0, keepdims=True)
        mu = sum_h / N
        var = sum_h2 / N - mu * mu
        scale = gamma_ref[...] * jax.lax.rsqrt(var + 1e-5)
        shift = beta_ref[...] - mu * scale
        scale_ref[...] = scale
        f_ref[...] = e * scale + shift

    @pl.when(jnp.logical_not(phase_a))
    def _apply():
        xb = xbuf_ref[pl.ds(t * R, R), :]
        a = jnp.dot(xb, w1_ref[:D, :], preferred_element_type=jnp.float32)
        seg_f = jax.lax.dot_general(
            oh_t, f_ref[...], (((0,), (0,)), ((), ())),
            preferred_element_type=jnp.float32)
        out_ref[...] = jax.nn.relu(a * scale_ref[...] + seg_f)


def kernel(p, x, o, W1, b1, gamma, beta, W2, b2):
    del p
    full = lambda shape: pl.BlockSpec(shape, lambda *_: (0,) * len(shape))
    x_spec = pl.BlockSpec((R, D), lambda i: (jnp.where(i < T, i, T - 1), 0))
    out_spec = pl.BlockSpec((R, D), lambda i: (jnp.where(i < T, 0, i - T), 0))

    return pl.pallas_call(
        _body,
        grid=(2 * T,),
        in_specs=[
            x_spec,
            full((B, 1)), full((2 * D, D)), full((1, D)), full((1, D)),
            full((1, D)), full((D, D)), full((1, D)),
        ],
        out_specs=out_spec,
        out_shape=jax.ShapeDtypeStruct((N, D), jnp.float32),
        scratch_shapes=[
            pltpu.VMEM((D, D), jnp.float32),
            pltpu.VMEM((B, D), jnp.float32),
            pltpu.VMEM((1, D), jnp.float32),
            pltpu.VMEM((B, D), jnp.float32),
            pltpu.VMEM((N, D), jnp.float32),
        ],
    )(x, o.reshape(B, 1), W1, b1.reshape(1, D), gamma.reshape(1, D),
      beta.reshape(1, D), W2, b2.reshape(1, D))


# speedup vs baseline: 14.9290x; 1.0370x over previous
import jax
import jax.numpy as jnp
from jax.experimental import pallas as pl
from jax.experimental.pallas import tpu as pltpu

N = 32768
B = 16
D = 128
R = 4096
T = N // R


def _body(x_ref, o_ref, w1_ref, b1_ref, gamma_ref, beta_ref, w2_ref, b2_ref,
          out_ref, gram_ref, segsum_ref, scale_ref, f_ref, xbuf_ref,
          ohbuf_ref):
    i = pl.program_id(0)
    phase_a = i < T
    t = jnp.where(phase_a, i, i - T)

    o_col = o_ref[...]
    op_col = jnp.concatenate(
        [jnp.zeros((1, 1), jnp.int32), o_col[:-1, :]], axis=0)

    @pl.when(phase_a)
    def _accum():
        @pl.when(i == 0)
        def _init():
            gram_ref[...] = jnp.zeros_like(gram_ref)
            segsum_ref[...] = jnp.zeros_like(segsum_ref)

        base = i * R
        r = jax.lax.broadcasted_iota(jnp.int32, (B, R), 1)
        oh_t = ((r >= op_col - base) & (r < o_col - base)).astype(jnp.float32)
        ohbuf_ref[:, pl.ds(i * R, R)] = oh_t
        x = x_ref[...]
        xbuf_ref[pl.ds(i * R, R), :] = x
        gram_ref[...] += jax.lax.dot_general(
            x, x, (((0,), (0,)), ((), ())), preferred_element_type=jnp.float32)
        segsum_ref[...] += jnp.dot(oh_t, x, preferred_element_type=jnp.float32)

    @pl.when(i == T)
    def _mid():
        cnt = (o_col - op_col).astype(jnp.float32)
        segsum = segsum_ref[...]
        w1a = w1_ref[:D, :]
        seg_mean = segsum / jnp.maximum(cnt, 1.0)
        g = jax.nn.relu(jnp.dot(seg_mean, w2_ref[...],
                                preferred_element_type=jnp.float32)
                        + b2_ref[...])
        e = jnp.dot(g, w1_ref[D:, :],
                    preferred_element_type=jnp.float32) + b1_ref[...]
        segsum_a = jnp.dot(segsum, w1a, preferred_element_type=jnp.float32)
        sum_a2 = jnp.sum(w1a * jnp.dot(gram_ref[...], w1a,
                                       preferred_element_type=jnp.float32),
                         axis=0, keepdims=True)
        sum_h = jnp.sum(segsum_a + cnt * e, axis=0, keepdims=True)
        sum_h2 = sum_a2 + jnp.sum(2.0 * e * segsum_a + cnt * e * e,
                                  axis=0, keepdims=True)
        mu = sum_h / N
        var = sum_h2 / N - mu * mu
        scale = gamma_ref[...] * jax.lax.rsqrt(var + 1e-5)
        shift = beta_ref[...] - mu * scale
        scale_ref[...] = scale
        f_ref[...] = e * scale + shift

    @pl.when(jnp.logical_not(phase_a))
    def _apply():
        xb = xbuf_ref[pl.ds(t * R, R), :]
        a = jnp.dot(xb, w1_ref[:D, :], preferred_element_type=jnp.float32)
        seg_f = jax.lax.dot_general(
            ohbuf_ref[:, pl.ds(t * R, R)], f_ref[...], (((0,), (0,)), ((), ())),
            preferred_element_type=jnp.float32)
        out_ref[...] = jax.nn.relu(a * scale_ref[...] + seg_f)


def kernel(p, x, o, W1, b1, gamma, beta, W2, b2):
    del p
    full = lambda shape: pl.BlockSpec(shape, lambda *_: (0,) * len(shape))
    x_spec = pl.BlockSpec((R, D), lambda i: (jnp.where(i < T, i, T - 1), 0))
    out_spec = pl.BlockSpec((R, D), lambda i: (jnp.where(i < T, 0, i - T), 0))

    return pl.pallas_call(
        _body,
        grid=(2 * T,),
        in_specs=[
            x_spec,
            full((B, 1)), full((2 * D, D)), full((1, D)), full((1, D)),
            full((1, D)), full((D, D)), full((1, D)),
        ],
        out_specs=out_spec,
        out_shape=jax.ShapeDtypeStruct((N, D), jnp.float32),
        scratch_shapes=[
            pltpu.VMEM((D, D), jnp.float32),
            pltpu.VMEM((B, D), jnp.float32),
            pltpu.VMEM((1, D), jnp.float32),
            pltpu.VMEM((B, D), jnp.float32),
            pltpu.VMEM((N, D), jnp.float32),
            pltpu.VMEM((B, N), jnp.float32),
        ],
    )(x, o.reshape(B, 1), W1, b1.reshape(1, D), gamma.reshape(1, D),
      beta.reshape(1, D), W2, b2.reshape(1, D))


# speedup vs baseline: 17.3507x; 1.1622x over previous
import jax
import jax.numpy as jnp
from jax.experimental import pallas as pl
from jax.experimental.pallas import tpu as pltpu

N = 32768
B = 16
D = 128
R = 8192
T = N // R


def _body(x_ref, o_ref, w1_ref, b1_ref, gamma_ref, beta_ref, w2_ref, b2_ref,
          out_ref, gram_ref, segsum_ref, scale_ref, f_ref, xbuf_ref,
          ohbuf_ref):
    i = pl.program_id(0)
    phase_a = i < T
    t = jnp.where(phase_a, i, i - T)

    o_col = o_ref[...]
    op_col = jnp.concatenate(
        [jnp.zeros((1, 1), jnp.int32), o_col[:-1, :]], axis=0)

    @pl.when(phase_a)
    def _accum():
        @pl.when(i == 0)
        def _init():
            gram_ref[...] = jnp.zeros_like(gram_ref)
            segsum_ref[...] = jnp.zeros_like(segsum_ref)

        base = i * R
        r = jax.lax.broadcasted_iota(jnp.int32, (B, R), 1)
        oh_t = ((r >= op_col - base) & (r < o_col - base)).astype(jnp.float32)
        ohbuf_ref[:, pl.ds(i * R, R)] = oh_t
        x = x_ref[...]
        xbuf_ref[pl.ds(i * R, R), :] = x
        gram_ref[...] += jax.lax.dot_general(
            x, x, (((0,), (0,)), ((), ())), preferred_element_type=jnp.float32)
        segsum_ref[...] += jnp.dot(oh_t, x, preferred_element_type=jnp.float32)

    @pl.when(i == T)
    def _mid():
        cnt = (o_col - op_col).astype(jnp.float32)
        segsum = segsum_ref[...]
        w1a = w1_ref[:D, :]
        seg_mean = segsum / jnp.maximum(cnt, 1.0)
        g = jax.nn.relu(jnp.dot(seg_mean, w2_ref[...],
                                preferred_element_type=jnp.float32)
                        + b2_ref[...])
        e = jnp.dot(g, w1_ref[D:, :],
                    preferred_element_type=jnp.float32) + b1_ref[...]
        segsum_a = jnp.dot(segsum, w1a, preferred_element_type=jnp.float32)
        sum_a2 = jnp.sum(w1a * jnp.dot(gram_ref[...], w1a,
                                       preferred_element_type=jnp.float32),
                         axis=0, keepdims=True)
        sum_h = jnp.sum(segsum_a + cnt * e, axis=0, keepdims=True)
        sum_h2 = sum_a2 + jnp.sum(2.0 * e * segsum_a + cnt * e * e,
                                  axis=0, keepdims=True)
        mu = sum_h / N
        var = sum_h2 / N - mu * mu
        scale = gamma_ref[...] * jax.lax.rsqrt(var + 1e-5)
        shift = beta_ref[...] - mu * scale
        scale_ref[...] = scale
        f_ref[...] = e * scale + shift

    @pl.when(jnp.logical_not(phase_a))
    def _apply():
        xb = xbuf_ref[pl.ds(t * R, R), :]
        a = jnp.dot(xb, w1_ref[:D, :], preferred_element_type=jnp.float32)
        seg_f = jax.lax.dot_general(
            ohbuf_ref[:, pl.ds(t * R, R)], f_ref[...], (((0,), (0,)), ((), ())),
            preferred_element_type=jnp.float32)
        out_ref[...] = jax.nn.relu(a * scale_ref[...] + seg_f)


def kernel(p, x, o, W1, b1, gamma, beta, W2, b2):
    del p
    full = lambda shape: pl.BlockSpec(shape, lambda *_: (0,) * len(shape))
    x_spec = pl.BlockSpec((R, D), lambda i: (jnp.where(i < T, i, T - 1), 0))
    out_spec = pl.BlockSpec((R, D), lambda i: (jnp.where(i < T, 0, i - T), 0))

    return pl.pallas_call(
        _body,
        grid=(2 * T,),
        in_specs=[
            x_spec,
            full((B, 1)), full((2 * D, D)), full((1, D)), full((1, D)),
            full((1, D)), full((D, D)), full((1, D)),
        ],
        out_specs=out_spec,
        out_shape=jax.ShapeDtypeStruct((N, D), jnp.float32),
        scratch_shapes=[
            pltpu.VMEM((D, D), jnp.float32),
            pltpu.VMEM((B, D), jnp.float32),
            pltpu.VMEM((1, D), jnp.float32),
            pltpu.VMEM((B, D), jnp.float32),
            pltpu.VMEM((N, D), jnp.float32),
            pltpu.VMEM((B, N), jnp.float32),
        ],
    )(x, o.reshape(B, 1), W1, b1.reshape(1, D), gamma.reshape(1, D),
      beta.reshape(1, D), W2, b2.reshape(1, D))


# speedup vs baseline: 17.3601x; 1.0005x over previous
import jax
import jax.numpy as jnp
from jax.experimental import pallas as pl
from jax.experimental.pallas import tpu as pltpu

N = 32768
B = 16
D = 128
R = 16384
T = N // R


def _body(x_ref, o_ref, w1_ref, b1_ref, gamma_ref, beta_ref, w2_ref, b2_ref,
          out_ref, gram_ref, segsum_ref, scale_ref, f_ref, xbuf_ref,
          ohbuf_ref):
    i = pl.program_id(0)
    phase_a = i < T
    t = jnp.where(phase_a, i, i - T)

    o_col = o_ref[...]
    op_col = jnp.concatenate(
        [jnp.zeros((1, 1), jnp.int32), o_col[:-1, :]], axis=0)

    @pl.when(phase_a)
    def _accum():
        @pl.when(i == 0)
        def _init():
            gram_ref[...] = jnp.zeros_like(gram_ref)
            segsum_ref[...] = jnp.zeros_like(segsum_ref)

        base = i * R
        r = jax.lax.broadcasted_iota(jnp.int32, (B, R), 1)
        oh_t = ((r >= op_col - base) & (r < o_col - base)).astype(jnp.float32)
        ohbuf_ref[:, pl.ds(i * R, R)] = oh_t
        x = x_ref[...]
        xbuf_ref[pl.ds(i * R, R), :] = x
        gram_ref[...] += jax.lax.dot_general(
            x, x, (((0,), (0,)), ((), ())), preferred_element_type=jnp.float32)
        segsum_ref[...] += jnp.dot(oh_t, x, preferred_element_type=jnp.float32)

    @pl.when(i == T)
    def _mid():
        cnt = (o_col - op_col).astype(jnp.float32)
        segsum = segsum_ref[...]
        w1a = w1_ref[:D, :]
        seg_mean = segsum / jnp.maximum(cnt, 1.0)
        g = jax.nn.relu(jnp.dot(seg_mean, w2_ref[...],
                                preferred_element_type=jnp.float32)
                        + b2_ref[...])
        e = jnp.dot(g, w1_ref[D:, :],
                    preferred_element_type=jnp.float32) + b1_ref[...]
        segsum_a = jnp.dot(segsum, w1a, preferred_element_type=jnp.float32)
        sum_a2 = jnp.sum(w1a * jnp.dot(gram_ref[...], w1a,
                                       preferred_element_type=jnp.float32),
                         axis=0, keepdims=True)
        sum_h = jnp.sum(segsum_a + cnt * e, axis=0, keepdims=True)
        sum_h2 = sum_a2 + jnp.sum(2.0 * e * segsum_a + cnt * e * e,
                                  axis=0, keepdims=True)
        mu = sum_h / N
        var = sum_h2 / N - mu * mu
        scale = gamma_ref[...] * jax.lax.rsqrt(var + 1e-5)
        shift = beta_ref[...] - mu * scale
        scale_ref[...] = scale
        f_ref[...] = e * scale + shift

    @pl.when(jnp.logical_not(phase_a))
    def _apply():
        xb = xbuf_ref[pl.ds(t * R, R), :]
        a = jnp.dot(xb, w1_ref[:D, :], preferred_element_type=jnp.float32)
        seg_f = jax.lax.dot_general(
            ohbuf_ref[:, pl.ds(t * R, R)], f_ref[...], (((0,), (0,)), ((), ())),
            preferred_element_type=jnp.float32)
        out_ref[...] = jax.nn.relu(a * scale_ref[...] + seg_f)


def kernel(p, x, o, W1, b1, gamma, beta, W2, b2):
    del p
    full = lambda shape: pl.BlockSpec(shape, lambda *_: (0,) * len(shape))
    x_spec = pl.BlockSpec((R, D), lambda i: (jnp.where(i < T, i, T - 1), 0))
    out_spec = pl.BlockSpec((R, D), lambda i: (jnp.where(i < T, 0, i - T), 0))

    return pl.pallas_call(
        _body,
        grid=(2 * T,),
        in_specs=[
            x_spec,
            full((B, 1)), full((2 * D, D)), full((1, D)), full((1, D)),
            full((1, D)), full((D, D)), full((1, D)),
        ],
        out_specs=out_spec,
        out_shape=jax.ShapeDtypeStruct((N, D), jnp.float32),
        scratch_shapes=[
            pltpu.VMEM((D, D), jnp.float32),
            pltpu.VMEM((B, D), jnp.float32),
            pltpu.VMEM((1, D), jnp.float32),
            pltpu.VMEM((B, D), jnp.float32),
            pltpu.VMEM((N, D), jnp.float32),
            pltpu.VMEM((B, N), jnp.float32),
        ],
    )(x, o.reshape(B, 1), W1, b1.reshape(1, D), gamma.reshape(1, D),
      beta.reshape(1, D), W2, b2.reshape(1, D))
